# baseline (speedup 1.0000x reference)
import numpy as np
import jax
import jax.numpy as jnp
from jax.experimental import pallas as pl
from jax.experimental.pallas import tpu as pltpu

_C_IN = 3
_C1 = 4
_C2 = 4
_K = 5
_H = 32
_H1 = 28
_P1 = 14
_H2 = 10
_P2 = 5
_F = _C2 * _P2 * _P2
_N_B = 1024
_G = 4

_NG = _H1 // _G
_M1 = 2 * _G * _P1
_KS1 = (_G + _K - 1) * _H
_M2 = 2 * _H2 * _P2
_KS2 = _P1 * _P1


def _factors():
    f32 = np.float32
    dy = np.arange(_K)
    a1 = (np.arange(_G + _K - 1)[None, :, None]
          == np.arange(_G)[None, None, :] + dy[:, None, None]).astype(f32)
    b1 = (np.arange(_H)[None, :, None, None]
          == 2 * np.arange(_P1)[None, None, None, :]
          + np.arange(2)[None, None, :, None]
          + dy[:, None, None, None]).astype(f32)
    f1 = np.einsum("djr,eupx->deprxju", a1, b1).reshape(_K * _K, _M1, _KS1)
    a2 = (np.arange(_P1)[None, :, None]
          == np.arange(_H2)[None, None, :] + dy[:, None, None]).astype(f32)
    b2 = (np.arange(_P1)[None, :, None, None]
          == 2 * np.arange(_P2)[None, None, None, :]
          + np.arange(2)[None, None, :, None]
          + dy[:, None, None, None]).astype(f32)
    f2 = np.einsum("dyr,eupx->deprxyu", a2, b2).reshape(_K * _K, _M2, _KS2)
    return f1, f2


_F1, _F2 = _factors()


def _encoder_body(x_ref, w1_ref, b1_ref, w2_ref, b2_ref,
                  wfc_ref, bfc_ref, wh_ref, bh_ref, out_ref, p1_ref):
    f32 = jnp.float32

    for g in range(_NG):
        xs = jnp.concatenate(
            [x_ref[c * (_H * _H) + g * _G * _H:
                   c * (_H * _H) + g * _G * _H + _KS1, :]
             for c in range(_C_IN)], axis=0)
        h = jnp.dot(w1_ref[...], xs,
                    preferred_element_type=f32)
        for o in range(_C1):
            ho = h[o * _M1:(o + 1) * _M1, :]
            vo = jnp.maximum(
                jnp.maximum(ho[:_M1 // 2, :], ho[_M1 // 2:, :]) + b1_ref[o],
                0.0)
            r0 = o * (_P1 * _P1) + 2 * g * _P1
            p1_ref[r0:r0 + _P1, :] = (
                jnp.maximum(vo[0 * _P1:1 * _P1], vo[1 * _P1:2 * _P1]))
            p1_ref[r0 + _P1:r0 + 2 * _P1, :] = (
                jnp.maximum(vo[2 * _P1:3 * _P1], vo[3 * _P1:4 * _P1]))

    fs = []
    for o in range(_C2):
        h2 = None
        for c in range(_C1):
            d = jnp.dot(w2_ref[o, c], p1_ref[c * _KS2:(c + 1) * _KS2, :],
                        preferred_element_type=f32)
            h2 = d if h2 is None else h2 + d
        vo = jnp.maximum(
            jnp.maximum(h2[:_M2 // 2, :], h2[_M2 // 2:, :]) + b2_ref[o],
            0.0)
        for k in range(_P2):
            fs.append(jnp.maximum(vo[(2 * k) * _P2:(2 * k + 1) * _P2],
                                  vo[(2 * k + 1) * _P2:(2 * k + 2) * _P2]))
    f = jnp.concatenate(fs, axis=0)

    hid = jnp.dot(wfc_ref[...], f, preferred_element_type=f32) + bfc_ref[...]
    hid = jnp.maximum(hid, 0.0)
    out_ref[...] = (jnp.dot(wh_ref[...], hid, preferred_element_type=f32)
                    + bh_ref[...])


def kernel(state, w1, b1, w2, b2, fcw, fcb, muw, mub, vaw, vab):
    f32 = jnp.float32
    in_shape = state.shape
    xt = state.astype(f32).reshape(-1, _C_IN * _H * _H).T
    B = xt.shape[1]
    L = muw.shape[0]

    nt = pl.cdiv(B, _N_B)
    bp = nt * _N_B
    if bp != B:
        xt = jnp.pad(xt, ((0, 0), (0, bp - B)))

    w1g = (w1.astype(f32).reshape(_C1 * _C_IN, _K * _K)
           @ jnp.asarray(_F1.reshape(_K * _K, -1))
           ).reshape(_C1, _C_IN, _M1, _KS1).transpose(0, 2, 1, 3) \
        .reshape(_C1 * _M1, _C_IN * _KS1)
    w2g = (w2.astype(f32).reshape(_C2 * _C2, _K * _K)
           @ jnp.asarray(_F2.reshape(_K * _K, -1))
           ).reshape(_C2, _C2, _M2, _KS2)

    wh = jnp.concatenate([muw, vaw], axis=0).astype(f32)
    bhr = jnp.concatenate([mub, vab]).astype(f32)[:, None]
    bfcr = fcb.astype(f32)[:, None]

    smem = pl.BlockSpec(memory_space=pltpu.MemorySpace.SMEM)
    full2 = lambda t: (0, 0)
    out = pl.pallas_call(
        _encoder_body,
        grid=(nt,),
        in_specs=[
            pl.BlockSpec((_C_IN * _H * _H, _N_B), lambda t: (0, t)),
            pl.BlockSpec((_C1 * _M1, _C_IN * _KS1), full2),
            smem,
            pl.BlockSpec((_C2, _C1, _M2, _KS2), lambda t: (0, 0, 0, 0)),
            smem,
            pl.BlockSpec((_F, _F), full2),
            pl.BlockSpec((_F, 1), full2),
            pl.BlockSpec((2 * L, _F), full2),
            pl.BlockSpec((2 * L, 1), full2),
        ],
        out_specs=pl.BlockSpec((2 * L, _N_B), lambda t: (0, t)),
        out_shape=jax.ShapeDtypeStruct((2 * L, bp), f32),
        scratch_shapes=[pltpu.VMEM((_C1 * _KS2, _N_B), f32)],
        compiler_params=pltpu.CompilerParams(
            dimension_semantics=("parallel",),
            vmem_limit_bytes=40 * 1024 * 1024),
    )(xt, w1g, b1.astype(f32), w2g, b2.astype(f32),
      fcw.astype(f32), bfcr, wh, bhr)

    mu = out[:L, :B].T.reshape(*in_shape[:-3], L)
    log_var = out[L:, :B].T.reshape(*in_shape[:-3], L)
    return mu, log_var
